# baseline (speedup 1.0000x reference)
import functools

import jax
import jax.numpy as jnp
from jax.experimental import pallas as pl
from jax.experimental.pallas import tpu as pltpu

_N = 4
_P = 4096
_BP = 1024
_R = _P // _BP


def _chamfer_kernel(p1_ref, p2_ref, out_ref, colmin_ref, acc_ref):
    n = pl.program_id(0)
    r = pl.program_id(1)

    a = p1_ref[0]
    b = p2_ref[0]

    ab = jax.lax.dot_general(
        a, b, (((1,), (1,)), ((), ())), preferred_element_type=jnp.float32
    )
    a2 = jnp.sum(a * a, axis=1, keepdims=True)
    b2 = jnp.sum(b * b, axis=1, keepdims=True).T

    d = a2 + b2 - 2.0 * ab

    row_min = jnp.min(d, axis=1)
    col_min = jnp.min(d, axis=0, keepdims=True)

    @pl.when(jnp.logical_and(n == 0, r == 0))
    def _init_acc():
        acc_ref[...] = jnp.zeros((1, 1), jnp.float32)

    @pl.when(r == 0)
    def _init_colmin():
        colmin_ref[...] = col_min

    @pl.when(r > 0)
    def _merge_colmin():
        colmin_ref[...] = jnp.minimum(colmin_ref[...], col_min)

    acc_ref[...] += jnp.sum(row_min)[None, None]

    @pl.when(r == _R - 1)
    def _fold_colmin():
        acc_ref[...] += jnp.sum(colmin_ref[...])[None, None]

    @pl.when(jnp.logical_and(n == _N - 1, r == _R - 1))
    def _finalize():
        out_ref[...] = acc_ref[...] * (1.0 / (_N * _P))


@jax.jit
def kernel(p1, p2):
    p1p = jnp.pad(p1, ((0, 0), (0, 0), (0, 5)))
    p2p = jnp.pad(p2, ((0, 0), (0, 0), (0, 5)))

    out = pl.pallas_call(
        _chamfer_kernel,
        grid=(_N, _R),
        in_specs=[
            pl.BlockSpec((1, _BP, 8), lambda n, r: (n, r, 0)),
            pl.BlockSpec((1, _P, 8), lambda n, r: (n, 0, 0)),
        ],
        out_specs=pl.BlockSpec((1, 1), lambda n, r: (0, 0)),
        out_shape=jax.ShapeDtypeStruct((1, 1), jnp.float32),
        scratch_shapes=[
            pltpu.VMEM((1, _P), jnp.float32),
            pltpu.VMEM((1, 1), jnp.float32),
        ],
    )(p1p, p2p)
    return out[0, 0]


# speedup vs baseline: 1.4451x; 1.4451x over previous
import functools

import jax
import jax.numpy as jnp
from jax.experimental import pallas as pl
from jax.experimental.pallas import tpu as pltpu

_N = 4
_P = 4096
_BP = 1024
_R = _P // _BP


def _chamfer_kernel(p1_ref, p2_ref, out_ref, colmin_ref, acc_ref):
    n = pl.program_id(0)
    r = pl.program_id(1)

    a = p1_ref[0]
    b = p2_ref[0]

    a2 = jnp.sum(a * a, axis=1, keepdims=True)
    b2 = jnp.sum(b * b, axis=1, keepdims=True)
    la = jax.lax.broadcasted_iota(jnp.int32, a.shape, 1)
    lb = jax.lax.broadcasted_iota(jnp.int32, b.shape, 1)
    a_aug = jnp.where(la < 3, -2.0 * a, jnp.where(la == 3, a2, (la == 4).astype(jnp.float32)))
    b_aug = jnp.where(lb < 3, b, jnp.where(lb == 4, b2, (lb == 3).astype(jnp.float32)))

    d = jax.lax.dot_general(
        a_aug, b_aug, (((1,), (1,)), ((), ())), preferred_element_type=jnp.float32
    )

    row_min = jnp.min(d, axis=1)
    col_min = jnp.min(d, axis=0, keepdims=True)

    @pl.when(jnp.logical_and(n == 0, r == 0))
    def _init_acc():
        acc_ref[...] = jnp.zeros((1, 1), jnp.float32)

    @pl.when(r == 0)
    def _init_colmin():
        colmin_ref[...] = col_min

    @pl.when(r > 0)
    def _merge_colmin():
        colmin_ref[...] = jnp.minimum(colmin_ref[...], col_min)

    acc_ref[...] += jnp.sum(row_min)[None, None]

    @pl.when(r == _R - 1)
    def _fold_colmin():
        acc_ref[...] += jnp.sum(colmin_ref[...])[None, None]

    @pl.when(jnp.logical_and(n == _N - 1, r == _R - 1))
    def _finalize():
        out_ref[...] = acc_ref[...] * (1.0 / (_N * _P))


@jax.jit
def kernel(p1, p2):
    p1p = jnp.pad(p1, ((0, 0), (0, 0), (0, 5)))
    p2p = jnp.pad(p2, ((0, 0), (0, 0), (0, 5)))

    out = pl.pallas_call(
        _chamfer_kernel,
        grid=(_N, _R),
        in_specs=[
            pl.BlockSpec((1, _BP, 8), lambda n, r: (n, r, 0)),
            pl.BlockSpec((1, _P, 8), lambda n, r: (n, 0, 0)),
        ],
        out_specs=pl.BlockSpec((1, 1), lambda n, r: (0, 0)),
        out_shape=jax.ShapeDtypeStruct((1, 1), jnp.float32),
        scratch_shapes=[
            pltpu.VMEM((1, _P), jnp.float32),
            pltpu.VMEM((1, 1), jnp.float32),
        ],
    )(p1p, p2p)
    return out[0, 0]
